# baseline (speedup 1.0000x reference)
import jax
import jax.numpy as jnp
from jax import lax
from jax.experimental import pallas as pl
from jax.experimental.pallas import tpu as pltpu
from jax.experimental.pallas import tpu_sc as plsc

B = 16384
H = 26
D = 32
N = B * H
NC, NS, L = 2, 16, 16
NW = NC * NS
CHUNK = 26 * 64
CPW = N // (NW * CHUNK)
GRAN = 128


def _sc_gather(flat_ids, pattern, table):
    mesh = plsc.VectorSubcoreMesh(core_axis_name="c", subcore_axis_name="s")

    @pl.kernel(
        mesh=mesh,
        out_type=jax.ShapeDtypeStruct((N, D), jnp.float32),
        scratch_types=[
            pltpu.VMEM((CHUNK,), jnp.int32),
            pltpu.VMEM((CHUNK,), jnp.int32),
            pltpu.VMEM((CHUNK, D), jnp.float32),
            pltpu.SemaphoreType.DMA,
        ],
        compiler_params=pltpu.CompilerParams(use_tc_tiling_on_sc=False),
    )
    def body(ids_hbm, pat_hbm, table_hbm, out_hbm, pat_v, idx_v, rows_v, sem):
        wid = lax.axis_index("s") * NC + lax.axis_index("c")
        pltpu.sync_copy(pat_hbm, pat_v)

        @pl.loop(0, CPW)
        def _chunk(ci):
            base = (wid * CPW + ci) * CHUNK
            pltpu.sync_copy(ids_hbm.at[pl.ds(base, CHUNK)], idx_v)

            @pl.loop(0, CHUNK // L)
            def _add(i):
                sl = pl.ds(i * L, L)
                idx_v[sl] = idx_v[sl] + pat_v[sl]

            @pl.loop(0, CHUNK // GRAN)
            def _gather(j):
                sl = pl.ds(j * GRAN, GRAN)
                pltpu.async_copy(
                    table_hbm.at[idx_v.at[sl]], rows_v.at[sl, :], sem
                ).wait()

            pltpu.sync_copy(rows_v, out_hbm.at[pl.ds(base, CHUNK)])

    return body(flat_ids, pattern, table)


def kernel(input_ids, offsets, table):
    flat_ids = input_ids.reshape(N)
    pattern = jnp.tile(offsets, CHUNK // H)
    out = _sc_gather(flat_ids, pattern, table)
    return out.reshape(B, H, D)


# speedup vs baseline: 1.0434x; 1.0434x over previous
import jax
import jax.numpy as jnp
from jax import lax
from jax.experimental import pallas as pl
from jax.experimental.pallas import tpu as pltpu
from jax.experimental.pallas import tpu_sc as plsc

B = 16384
H = 26
D = 32
N = B * H
NC, NS, L = 2, 16, 16
NW = NC * NS
CHUNK = 26 * 64
CPW = N // (NW * CHUNK)
NPW = CPW * CHUNK
GRAN = 128
NG = CHUNK // GRAN


def _sc_gather(flat_ids, pattern, table):
    mesh = plsc.VectorSubcoreMesh(core_axis_name="c", subcore_axis_name="s")

    @pl.kernel(
        mesh=mesh,
        out_type=jax.ShapeDtypeStruct((N, D), jnp.float32),
        scratch_types=[
            pltpu.VMEM((CHUNK,), jnp.int32),
            pltpu.VMEM((NPW,), jnp.int32),
            pltpu.VMEM((CHUNK, D), jnp.float32),
            pltpu.VMEM((CHUNK, D), jnp.float32),
            pltpu.SemaphoreType.DMA,
            pltpu.SemaphoreType.DMA,
        ],
        compiler_params=pltpu.CompilerParams(use_tc_tiling_on_sc=False),
    )
    def body(ids_hbm, pat_hbm, table_hbm, out_hbm,
             pat_v, idx_v, rows0_v, rows1_v, sem_g, sem_out):
        wid = lax.axis_index("s") * NC + lax.axis_index("c")
        wbase = wid * NPW
        pltpu.sync_copy(pat_hbm, pat_v)
        pltpu.sync_copy(ids_hbm.at[pl.ds(wbase, NPW)], idx_v)

        @pl.loop(0, CPW)
        def _add_chunk(c):
            @pl.loop(0, CHUNK // L)
            def _add(i):
                dst = pl.ds(c * CHUNK + i * L, L)
                idx_v[dst] = idx_v[dst] + pat_v[pl.ds(i * L, L)]

        def fire_drain_store(c, rows_v):
            handles = []
            for j in range(NG):
                sl = pl.ds(c * CHUNK + j * GRAN, GRAN)
                handles.append(pltpu.make_async_copy(
                    table_hbm.at[idx_v.at[sl]],
                    rows_v.at[pl.ds(j * GRAN, GRAN), :], sem_g))
            for h in handles:
                h.start()
            for h in handles:
                h.wait()
            st = pltpu.make_async_copy(
                rows_v, out_hbm.at[pl.ds(wbase + c * CHUNK, CHUNK)], sem_out)
            st.start()
            return st

        def wait_store(rows_v):
            pltpu.make_async_copy(
                out_hbm.at[pl.ds(wbase, CHUNK)], rows_v, sem_out).wait()

        fire_drain_store(0, rows0_v)
        fire_drain_store(1, rows1_v)

        @pl.loop(2, CPW, step=2)
        def _chunk(c):
            wait_store(rows0_v)
            fire_drain_store(c, rows0_v)
            wait_store(rows1_v)
            fire_drain_store(c + 1, rows1_v)

        wait_store(rows0_v)
        wait_store(rows1_v)

    return body(flat_ids, pattern, table)


def kernel(input_ids, offsets, table):
    flat_ids = input_ids.reshape(N)
    pattern = jnp.tile(offsets, CHUNK // H)
    out = _sc_gather(flat_ids, pattern, table)
    return out.reshape(B, H, D)
